# baseline (speedup 1.0000x reference)
import functools

import jax
import jax.numpy as jnp
from jax import lax
from jax.experimental import pallas as pl
from jax.experimental.pallas import tpu as pltpu
from jax.experimental.pallas import tpu_sc as plsc

N = 4096
IMG = 64
BR = 512
NBLK = N // BR
INV_SQRT3 = float(1.0 / (3.0 ** 0.5))
NEG = float("-inf")


def _att_body(x_ref, wq_ref, wk_ref, bq_ref, colsum_ref, colors_ref):
    step = pl.program_id(0)
    xf = x_ref[...].astype(jnp.float32)
    x_blk = x_ref[pl.ds(step * BR, BR), :].astype(jnp.float32)
    wq = wq_ref[...]
    wk = wk_ref[...]
    bq = bq_ref[...]
    m33 = lax.dot_general(wq, wk, (((0,), (0,)), ((), ())),
                          precision=lax.Precision.HIGHEST)
    v3 = lax.dot_general(bq, wk, (((1,), (0,)), ((), ())),
                         precision=lax.Precision.HIGHEST)
    a = (lax.dot_general(x_blk, m33, (((1,), (0,)), ((), ())),
                         precision=lax.Precision.HIGHEST) + v3) * INV_SQRT3
    z = lax.dot_general(a, xf, (((1,), (1,)), ((), ())),
                        precision=lax.Precision.HIGHEST)
    zmax = jnp.max(z, axis=1, keepdims=True)
    e = jnp.exp(z - zmax)
    r = lax.dot_general(e, jnp.ones((N, 1), jnp.float32),
                        (((1,), (0,)), ((), ())),
                        precision=lax.Precision.HIGHEST)
    w = 1.0 / r
    part = lax.dot_general(w, e, (((0,), (0,)), ((), ())),
                           precision=lax.Precision.HIGHEST)

    @pl.when(step == 0)
    def _():
        colsum_ref[...] = part

    @pl.when(step != 0)
    def _():
        colsum_ref[...] += part

    colors_ref[...] = jnp.floor(jnp.mean(x_blk, axis=1, keepdims=True) / 255.0)


def _attention_colsum(x, wq, wk, bq_row):
    return pl.pallas_call(
        _att_body,
        grid=(NBLK,),
        in_specs=[
            pl.BlockSpec((N, 3), lambda i: (0, 0)),
            pl.BlockSpec((512, 3), lambda i: (0, 0)),
            pl.BlockSpec((512, 3), lambda i: (0, 0)),
            pl.BlockSpec((1, 512), lambda i: (0, 0)),
        ],
        out_specs=[
            pl.BlockSpec((1, N), lambda i: (0, 0)),
            pl.BlockSpec((BR, 1), lambda i: (i, 0)),
        ],
        out_shape=[
            jax.ShapeDtypeStruct((1, N), jnp.float32),
            jax.ShapeDtypeStruct((N, 1), jnp.float32),
        ],
    )(x, wq, wk, bq_row)


_GDN = lax.GatherDimensionNumbers(
    offset_dims=(), collapsed_slice_dims=(0,), start_index_map=(0,))


def _perm(x, idx16):
    return lax.gather(x, idx16.reshape(16, 1), _GDN, (1,),
                      mode=lax.GatherScatterMode.PROMISE_IN_BOUNDS)


def _sc_select_body(att_hbm, col_hbm, wct_hbm, bc_hbm, out_hbm,
                    att_v, col_v, wct_v, bc_v, out_v):
    cid = lax.axis_index("c")
    sid = lax.axis_index("s")

    @pl.when(jnp.logical_and(cid == 0, sid == 0))
    def _():
        pltpu.sync_copy(att_hbm, att_v)
        pltpu.sync_copy(col_hbm, col_v)
        pltpu.sync_copy(wct_hbm, wct_v)
        pltpu.sync_copy(bc_hbm, bc_v)
        lane = lax.iota(jnp.int32, 16)
        neg = jnp.full((16,), NEG, jnp.float32)

        def bfly(x, op):
            for c in (1, 2, 4, 8):
                x = op(x, _perm(x, lane ^ c))
            return x

        def find_one(k, topi):
            def scan_body(j, carry):
                bv, bi = carry
                base = j * 128
                for u in range(8):
                    v = att_v[pl.ds(base + u * 16, 16)]
                    idxs = base + u * 16 + lane
                    upd = v > bv
                    bv = jnp.where(upd, v, bv)
                    bi = jnp.where(upd, idxs, bi)
                return (bv, bi)

            bv, bi = lax.fori_loop(
                0, N // 128, scan_body,
                (neg, jnp.zeros((16,), jnp.int32)))
            mval = bfly(bv, jnp.maximum)
            idx = bfly(jnp.where(bv == mval, bi, N), jnp.minimum)
            plsc.store_scatter(att_v, [idx], neg, mask=lane == 0)
            return jnp.where(lane == k, idx, topi)

        topi = lax.fori_loop(0, 10, find_one, jnp.zeros((16,), jnp.int32))

        rows = topi // IMG
        cols = topi % IMG
        xf = (rows.astype(jnp.float32) + 0.5) / float(IMG)
        yf = (cols.astype(jnp.float32) + 0.5) / float(IMG)
        colr = plsc.load_gather(col_v, [topi])

        logits = bc_v[...]
        for t in range(30):
            pp, cc = t // 3, t % 3
            src = (xf, yf, colr)[cc]
            f_t = _perm(src, jnp.full((16,), pp, jnp.int32))
            logits = logits + f_t * wct_v[pl.ds(t * 16, 16)]

        mlog = bfly(logits, jnp.maximum)
        e = jnp.where(lane < 15, jnp.exp(logits - mlog), 0.0)
        s = bfly(e, jnp.add)
        actions = e / s
        ma = bfly(actions, jnp.maximum)
        sel = bfly(jnp.where(actions == ma, lane, 99), jnp.minimum)
        res = jnp.where(ma > 0.3, sel, 4)
        out_v[...] = jnp.where(lane == 0, res, 0)
        pltpu.sync_copy(out_v, out_hbm)


@functools.lru_cache(maxsize=1)
def _build_sc_select():
    mesh = plsc.VectorSubcoreMesh(core_axis_name="c", subcore_axis_name="s")

    @functools.partial(
        pl.kernel,
        out_type=jax.ShapeDtypeStruct((16,), jnp.int32),
        mesh=mesh,
        compiler_params=pltpu.CompilerParams(needs_layout_passes=False),
        scratch_types=[
            pltpu.VMEM((N,), jnp.float32),
            pltpu.VMEM((N,), jnp.float32),
            pltpu.VMEM((480,), jnp.float32),
            pltpu.VMEM((16,), jnp.float32),
            pltpu.VMEM((16,), jnp.int32),
        ],
    )
    def _sc_select(att_hbm, col_hbm, wct_hbm, bc_hbm, out_hbm,
                   att_v, col_v, wct_v, bc_v, out_v):
        _sc_select_body(att_hbm, col_hbm, wct_hbm, bc_hbm, out_hbm,
                        att_v, col_v, wct_v, bc_v, out_v)

    return _sc_select


def kernel(obs, Wq, bq, Wk, bk, Wc, bc):
    del bk
    x = obs.reshape(N, 3)
    bq_row = bq.reshape(1, 512)
    colsum, colors = _attention_colsum(x, Wq, Wk, bq_row)
    wct = jnp.pad(Wc.T, ((0, 0), (0, 1))).reshape(480)
    bcp = jnp.concatenate([bc, jnp.full((1,), -1e30, jnp.float32)])
    out16 = _build_sc_select()(colsum.reshape(N), colors.reshape(N), wct, bcp)
    return out16[:1]


# speedup vs baseline: 3.9576x; 3.9576x over previous
import functools

import jax
import jax.numpy as jnp
from jax import lax
from jax.experimental import pallas as pl
from jax.experimental.pallas import tpu as pltpu
from jax.experimental.pallas import tpu_sc as plsc

N = 4096
IMG = 64
BR = 512
NBLK = N // BR
INV_SQRT3 = float(1.0 / (3.0 ** 0.5))
NEG = float("-inf")


def _att_body(x_ref, xt_ref, wq_ref, wk_ref, bq_ref, colsum_ref):
    step = pl.program_id(0)
    x_blk = x_ref[...].astype(jnp.float32)
    wq = wq_ref[...]
    wk = wk_ref[...]
    bq = bq_ref[...]
    m33 = lax.dot_general(wq, wk, (((0,), (0,)), ((), ())),
                          precision=lax.Precision.HIGHEST)
    v3 = lax.dot_general(bq, wk, (((1,), (0,)), ((), ())),
                         precision=lax.Precision.HIGHEST)
    a = (lax.dot_general(x_blk, m33, (((1,), (0,)), ((), ())),
                         precision=lax.Precision.HIGHEST) + v3) * INV_SQRT3
    z = (a[:, 0:1] * xt_ref[0:1, :]
         + a[:, 1:2] * xt_ref[1:2, :]
         + a[:, 2:3] * xt_ref[2:3, :])
    zmax = jnp.max(z, axis=1, keepdims=True)
    e = jnp.exp(z - zmax)
    r = jnp.sum(e, axis=1, keepdims=True)
    ew = e * (1.0 / r)
    part = jnp.sum(ew, axis=0, keepdims=True)

    @pl.when(step == 0)
    def _():
        colsum_ref[...] = part

    @pl.when(step != 0)
    def _():
        colsum_ref[...] += part


def _attention_colsum(x, xt, wq, wk, bq_row):
    return pl.pallas_call(
        _att_body,
        grid=(NBLK,),
        in_specs=[
            pl.BlockSpec((BR, 3), lambda i: (i, 0)),
            pl.BlockSpec((3, N), lambda i: (0, 0)),
            pl.BlockSpec((512, 3), lambda i: (0, 0)),
            pl.BlockSpec((512, 3), lambda i: (0, 0)),
            pl.BlockSpec((1, 512), lambda i: (0, 0)),
        ],
        out_specs=pl.BlockSpec((1, N), lambda i: (0, 0)),
        out_shape=jax.ShapeDtypeStruct((1, N), jnp.float32),
    )(x, xt, wq, wk, bq_row)


_GDN = lax.GatherDimensionNumbers(
    offset_dims=(), collapsed_slice_dims=(0,), start_index_map=(0,))


def _perm(x, idx16):
    return lax.gather(x, idx16.reshape(16, 1), _GDN, (1,),
                      mode=lax.GatherScatterMode.PROMISE_IN_BOUNDS)


def _sc_select_body(att_hbm, obs_hbm, aux_hbm, out_hbm,
                    att_v, obs_v, aux_v, out_v):
    cid = lax.axis_index("c")
    sid = lax.axis_index("s")

    @pl.when(jnp.logical_and(cid == 0, sid == 0))
    def _():
        pltpu.sync_copy(att_hbm, att_v)
        pltpu.sync_copy(obs_hbm, obs_v)
        pltpu.sync_copy(aux_hbm, aux_v)
        lane = lax.iota(jnp.int32, 16)
        neg = jnp.full((16,), NEG, jnp.float32)

        def bfly(x, op):
            for c in (1, 2, 4, 8):
                x = op(x, _perm(x, lane ^ c))
            return x

        def find_one(k, topi):
            def scan_body(j, carry):
                bv, bi = carry
                base = j * 128
                for u in range(8):
                    v = att_v[pl.ds(base + u * 16, 16)]
                    idxs = base + u * 16 + lane
                    upd = v > bv
                    bv = jnp.where(upd, v, bv)
                    bi = jnp.where(upd, idxs, bi)
                return (bv, bi)

            bv, bi = lax.fori_loop(
                0, N // 128, scan_body,
                (neg, jnp.zeros((16,), jnp.int32)))
            mval = bfly(bv, jnp.maximum)
            idx = bfly(jnp.where(bv == mval, bi, N), jnp.minimum)
            plsc.store_scatter(att_v, [idx], neg, mask=lane == 0)
            return jnp.where(lane == k, idx, topi)

        topi = lax.fori_loop(0, 10, find_one, jnp.zeros((16,), jnp.int32))

        rows = topi // IMG
        cols = topi % IMG
        xf = (rows.astype(jnp.float32) + 0.5) / float(IMG)
        yf = (cols.astype(jnp.float32) + 0.5) / float(IMG)
        p0 = plsc.load_gather(obs_v, [topi * 3])
        p1 = plsc.load_gather(obs_v, [topi * 3 + 1])
        p2 = plsc.load_gather(obs_v, [topi * 3 + 2])
        colr = jnp.where(p0 + p1 + p2 == 765, 1.0, 0.0)

        logits = aux_v[pl.ds(480, 16)]
        for t in range(30):
            pp, cc = t // 3, t % 3
            src = (xf, yf, colr)[cc]
            f_t = _perm(src, jnp.full((16,), pp, jnp.int32))
            logits = logits + f_t * aux_v[pl.ds(t * 16, 16)]

        mlog = bfly(logits, jnp.maximum)
        e = jnp.where(lane < 15, jnp.exp(logits - mlog), 0.0)
        s = bfly(e, jnp.add)
        actions = e / s
        ma = bfly(actions, jnp.maximum)
        sel = bfly(jnp.where(actions == ma, lane, 99), jnp.minimum)
        res = jnp.where(ma > 0.3, sel, 4)
        out_v[...] = jnp.where(lane == 0, res, 0)
        pltpu.sync_copy(out_v, out_hbm)


@functools.lru_cache(maxsize=1)
def _build_sc_select():
    mesh = plsc.VectorSubcoreMesh(core_axis_name="c", subcore_axis_name="s")

    @functools.partial(
        pl.kernel,
        out_type=jax.ShapeDtypeStruct((16,), jnp.int32),
        mesh=mesh,
        compiler_params=pltpu.CompilerParams(needs_layout_passes=False),
        scratch_types=[
            pltpu.VMEM((N,), jnp.float32),
            pltpu.VMEM((3 * N,), jnp.int32),
            pltpu.VMEM((496,), jnp.float32),
            pltpu.VMEM((16,), jnp.int32),
        ],
    )
    def _sc_select(att_hbm, obs_hbm, aux_hbm, out_hbm, att_v, obs_v, aux_v,
                   out_v):
        _sc_select_body(att_hbm, obs_hbm, aux_hbm, out_hbm,
                        att_v, obs_v, aux_v, out_v)

    return _sc_select


def kernel(obs, Wq, bq, Wk, bk, Wc, bc):
    del bk
    x = obs.reshape(N, 3)
    xt = x.astype(jnp.float32).T
    bq_row = bq.reshape(1, 512)
    colsum = _attention_colsum(x, xt, Wq, Wk, bq_row)
    aux = jnp.concatenate([
        jnp.pad(Wc.T, ((0, 0), (0, 1))).reshape(480),
        bc, jnp.full((1,), -1e30, jnp.float32)])
    out16 = _build_sc_select()(colsum.reshape(N), obs.reshape(3 * N), aux)
    return out16[:1]


# speedup vs baseline: 4.0502x; 1.0234x over previous
import functools

import jax
import jax.numpy as jnp
from jax import lax
from jax.experimental import pallas as pl
from jax.experimental.pallas import tpu as pltpu
from jax.experimental.pallas import tpu_sc as plsc

N = 4096
IMG = 64
BR = 512
NBLK = N // BR
ZSCALE = float((1.0 / (3.0 ** 0.5)) * 1.4426950408889634)
NEG = float("-inf")


def _att_body(x_ref, xt_ref, wq_ref, wk_ref, bq_ref, colsum_ref):
    step = pl.program_id(0)
    x_blk = x_ref[...].astype(jnp.float32)
    wq = wq_ref[...]
    wk = wk_ref[...]
    bq = bq_ref[...]
    m33 = lax.dot_general(wq, wk, (((0,), (0,)), ((), ())),
                          precision=lax.Precision.HIGHEST)
    v3 = lax.dot_general(bq, wk, (((1,), (0,)), ((), ())),
                         precision=lax.Precision.HIGHEST)
    a = (lax.dot_general(x_blk, m33, (((1,), (0,)), ((), ())),
                         precision=lax.Precision.HIGHEST) + v3) * ZSCALE
    z = (a[:, 0:1] * xt_ref[0:1, :]
         + a[:, 1:2] * xt_ref[1:2, :]
         + a[:, 2:3] * xt_ref[2:3, :])
    zmax = jnp.max(z, axis=1, keepdims=True)
    e = jnp.exp2(z - zmax)
    r = jnp.sum(e, axis=1, keepdims=True)
    ew = e * (1.0 / r)
    part = jnp.sum(ew, axis=0, keepdims=True)

    @pl.when(step == 0)
    def _():
        colsum_ref[...] = part

    @pl.when(step != 0)
    def _():
        colsum_ref[...] += part


def _attention_colsum(x, xt, wq, wk, bq_row):
    return pl.pallas_call(
        _att_body,
        grid=(NBLK,),
        in_specs=[
            pl.BlockSpec((BR, 3), lambda i: (i, 0)),
            pl.BlockSpec((3, N), lambda i: (0, 0)),
            pl.BlockSpec((512, 3), lambda i: (0, 0)),
            pl.BlockSpec((512, 3), lambda i: (0, 0)),
            pl.BlockSpec((1, 512), lambda i: (0, 0)),
        ],
        out_specs=pl.BlockSpec((1, N), lambda i: (0, 0)),
        out_shape=jax.ShapeDtypeStruct((1, N), jnp.float32),
    )(x, xt, wq, wk, bq_row)


_GDN = lax.GatherDimensionNumbers(
    offset_dims=(), collapsed_slice_dims=(0,), start_index_map=(0,))


def _perm(x, idx16):
    return lax.gather(x, idx16.reshape(16, 1), _GDN, (1,),
                      mode=lax.GatherScatterMode.PROMISE_IN_BOUNDS)


def _sc_select_body(att_hbm, obs_hbm, aux_hbm, out_hbm,
                    att_v, obs_v, aux_v, out_v):
    cid = lax.axis_index("c")
    sid = lax.axis_index("s")

    @pl.when(jnp.logical_and(cid == 0, sid == 0))
    def _():
        pltpu.sync_copy(att_hbm, att_v)
        pltpu.sync_copy(obs_hbm, obs_v)
        pltpu.sync_copy(aux_hbm, aux_v)
        lane = lax.iota(jnp.int32, 16)
        neg = jnp.full((16,), NEG, jnp.float32)

        def bfly(x, op):
            for c in (1, 2, 4, 8):
                x = op(x, _perm(x, lane ^ c))
            return x

        zero16 = jnp.zeros((16,), jnp.int32)

        def find_one(k, topi):
            def scan_body(j, carry):
                bv, bi = carry
                base = j * 128
                for u in range(8):
                    v = att_v[0, pl.ds(base + u * 16, 16)]
                    idxs = base + u * 16 + lane
                    upd = v > bv
                    bv = jnp.where(upd, v, bv)
                    bi = jnp.where(upd, idxs, bi)
                return (bv, bi)

            bv, bi = lax.fori_loop(
                0, N // 128, scan_body, (neg, zero16))
            mval = bfly(bv, jnp.maximum)
            idx = bfly(jnp.where(bv == mval, bi, N), jnp.minimum)
            plsc.store_scatter(att_v, [zero16, idx], neg, mask=lane == 0)
            return jnp.where(lane == k, idx, topi)

        topi = lax.fori_loop(0, 10, find_one, jnp.zeros((16,), jnp.int32))

        rows = topi // IMG
        cols = topi % IMG
        xf = (rows.astype(jnp.float32) + 0.5) / float(IMG)
        yf = (cols.astype(jnp.float32) + 0.5) / float(IMG)
        p0 = plsc.load_gather(obs_v, [topi * 3])
        p1 = plsc.load_gather(obs_v, [topi * 3 + 1])
        p2 = plsc.load_gather(obs_v, [topi * 3 + 2])
        colr = jnp.where(p0 + p1 + p2 == 765, 1.0, 0.0)

        logits = aux_v[pl.ds(480, 16)]
        for t in range(30):
            pp, cc = t // 3, t % 3
            src = (xf, yf, colr)[cc]
            f_t = _perm(src, jnp.full((16,), pp, jnp.int32))
            logits = logits + f_t * aux_v[pl.ds(t * 16, 16)]

        mlog = bfly(logits, jnp.maximum)
        e = jnp.where(lane < 15, jnp.exp(logits - mlog), 0.0)
        s = bfly(e, jnp.add)
        actions = e / s
        ma = bfly(actions, jnp.maximum)
        sel = bfly(jnp.where(actions == ma, lane, 99), jnp.minimum)
        res = jnp.where(ma > 0.3, sel, 4)
        out_v[...] = jnp.where(lane == 0, res, 0)
        pltpu.sync_copy(out_v, out_hbm)


@functools.lru_cache(maxsize=1)
def _build_sc_select():
    mesh = plsc.VectorSubcoreMesh(core_axis_name="c", subcore_axis_name="s")

    @functools.partial(
        pl.kernel,
        out_type=jax.ShapeDtypeStruct((16,), jnp.int32),
        mesh=mesh,
        compiler_params=pltpu.CompilerParams(needs_layout_passes=False),
        scratch_types=[
            pltpu.VMEM((1, N), jnp.float32),
            pltpu.VMEM((3 * N,), jnp.int32),
            pltpu.VMEM((496,), jnp.float32),
            pltpu.VMEM((16,), jnp.int32),
        ],
    )
    def _sc_select(att_hbm, obs_hbm, aux_hbm, out_hbm, att_v, obs_v, aux_v,
                   out_v):
        _sc_select_body(att_hbm, obs_hbm, aux_hbm, out_hbm,
                        att_v, obs_v, aux_v, out_v)

    return _sc_select


def kernel(obs, Wq, bq, Wk, bk, Wc, bc):
    del bk
    x = obs.reshape(N, 3)
    xt = x.astype(jnp.float32).T
    bq_row = bq.reshape(1, 512)
    colsum = _attention_colsum(x, xt, Wq, Wk, bq_row)
    aux = jnp.concatenate([
        jnp.pad(Wc.T, ((0, 0), (0, 1))).reshape(480),
        bc, jnp.full((1,), -1e30, jnp.float32)])
    out16 = _build_sc_select()(colsum, obs.reshape(3 * N), aux)
    return out16[:1]


# speedup vs baseline: 4.2708x; 1.0545x over previous
import functools

import jax
import jax.numpy as jnp
from jax import lax
from jax.experimental import pallas as pl
from jax.experimental.pallas import tpu as pltpu
from jax.experimental.pallas import tpu_sc as plsc

N = 4096
IMG = 64
BR = 512
NBLK = N // BR
ZSCALE = float((1.0 / (3.0 ** 0.5)) * 1.4426950408889634)
NEG = float("-inf")


def _att_body(x_ref, xt_ref, wq_ref, wk_ref, bq_ref, colsum_ref):
    step = pl.program_id(0)
    x_blk = x_ref[...].astype(jnp.float32)
    wq = wq_ref[...]
    wk = wk_ref[...]
    bq = bq_ref[...]
    m33 = lax.dot_general(wq, wk, (((0,), (0,)), ((), ())),
                          precision=lax.Precision.HIGHEST)
    v3 = lax.dot_general(bq, wk, (((1,), (0,)), ((), ())),
                         precision=lax.Precision.HIGHEST)
    a = (lax.dot_general(x_blk, m33, (((1,), (0,)), ((), ())),
                         precision=lax.Precision.HIGHEST) + v3) * ZSCALE
    z = (a[:, 0:1] * xt_ref[0:1, :]
         + a[:, 1:2] * xt_ref[1:2, :]
         + a[:, 2:3] * xt_ref[2:3, :])
    zmax = jnp.max(z, axis=1, keepdims=True)
    e = jnp.exp2(z - zmax)
    r = jnp.sum(e, axis=1, keepdims=True)
    ew = e * (1.0 / r)
    part = jnp.sum(ew, axis=0, keepdims=True)

    @pl.when(step == 0)
    def _():
        s3 = xt_ref[0:1, :] + xt_ref[1:2, :] + xt_ref[2:3, :]
        colsum_ref[1:2, :] = jnp.where(s3 == 765.0, 1.0, 0.0)
        colsum_ref[0:1, :] = part

    @pl.when(step != 0)
    def _():
        colsum_ref[0:1, :] += part


def _attention_colsum(x, xt, wq, wk, bq_row):
    return pl.pallas_call(
        _att_body,
        grid=(NBLK,),
        in_specs=[
            pl.BlockSpec((BR, 3), lambda i: (i, 0)),
            pl.BlockSpec((3, N), lambda i: (0, 0)),
            pl.BlockSpec((512, 3), lambda i: (0, 0)),
            pl.BlockSpec((512, 3), lambda i: (0, 0)),
            pl.BlockSpec((1, 512), lambda i: (0, 0)),
        ],
        out_specs=pl.BlockSpec((2, N), lambda i: (0, 0)),
        out_shape=jax.ShapeDtypeStruct((2, N), jnp.float32),
    )(x, xt, wq, wk, bq_row)


_GDN = lax.GatherDimensionNumbers(
    offset_dims=(), collapsed_slice_dims=(0,), start_index_map=(0,))


def _perm(x, idx16):
    return lax.gather(x, idx16.reshape(16, 1), _GDN, (1,),
                      mode=lax.GatherScatterMode.PROMISE_IN_BOUNDS)


def _sc_select_body(att_hbm, aux_hbm, out_hbm, att_v, aux_v, out_v):
    cid = lax.axis_index("c")
    sid = lax.axis_index("s")

    @pl.when(jnp.logical_and(cid == 0, sid == 0))
    def _():
        pltpu.sync_copy(att_hbm, att_v)
        pltpu.sync_copy(aux_hbm, aux_v)
        lane = lax.iota(jnp.int32, 16)
        neg = jnp.full((16,), NEG, jnp.float32)

        def bfly(x, op):
            for c in (1, 2, 4, 8):
                x = op(x, _perm(x, lane ^ c))
            return x

        zero16 = jnp.zeros((16,), jnp.int32)

        def find_one(k, topi):
            def scan_body(j, carry):
                bv, bi = carry
                base = j * 128
                for u in range(8):
                    v = att_v[0, pl.ds(base + u * 16, 16)]
                    idxs = base + u * 16 + lane
                    upd = v > bv
                    bv = jnp.where(upd, v, bv)
                    bi = jnp.where(upd, idxs, bi)
                return (bv, bi)

            bv, bi = lax.fori_loop(
                0, N // 128, scan_body, (neg, zero16))
            mval = bfly(bv, jnp.maximum)
            idx = bfly(jnp.where(bv == mval, bi, N), jnp.minimum)
            plsc.store_scatter(att_v, [zero16, idx], neg, mask=lane == 0)
            return jnp.where(lane == k, idx, topi)

        topi = lax.fori_loop(0, 10, find_one, jnp.zeros((16,), jnp.int32))

        rows = topi // IMG
        cols = topi % IMG
        xf = (rows.astype(jnp.float32) + 0.5) / float(IMG)
        yf = (cols.astype(jnp.float32) + 0.5) / float(IMG)
        colr = plsc.load_gather(att_v, [zero16 + 1, topi])

        logits = aux_v[pl.ds(480, 16)]
        for t in range(30):
            pp, cc = t // 3, t % 3
            src = (xf, yf, colr)[cc]
            f_t = _perm(src, jnp.full((16,), pp, jnp.int32))
            logits = logits + f_t * aux_v[pl.ds(t * 16, 16)]

        mlog = bfly(logits, jnp.maximum)
        e = jnp.where(lane < 15, jnp.exp(logits - mlog), 0.0)
        s = bfly(e, jnp.add)
        actions = e / s
        ma = bfly(actions, jnp.maximum)
        sel = bfly(jnp.where(actions == ma, lane, 99), jnp.minimum)
        res = jnp.where(ma > 0.3, sel, 4)
        out_v[...] = jnp.where(lane == 0, res, 0)
        pltpu.sync_copy(out_v, out_hbm)


@functools.lru_cache(maxsize=1)
def _build_sc_select():
    mesh = plsc.VectorSubcoreMesh(core_axis_name="c", subcore_axis_name="s")

    @functools.partial(
        pl.kernel,
        out_type=jax.ShapeDtypeStruct((16,), jnp.int32),
        mesh=mesh,
        compiler_params=pltpu.CompilerParams(needs_layout_passes=False),
        scratch_types=[
            pltpu.VMEM((2, N), jnp.float32),
            pltpu.VMEM((496,), jnp.float32),
            pltpu.VMEM((16,), jnp.int32),
        ],
    )
    def _sc_select(att_hbm, aux_hbm, out_hbm, att_v, aux_v, out_v):
        _sc_select_body(att_hbm, aux_hbm, out_hbm, att_v, aux_v, out_v)

    return _sc_select


def kernel(obs, Wq, bq, Wk, bk, Wc, bc):
    del bk
    x = obs.reshape(N, 3)
    xt = x.astype(jnp.float32).T
    bq_row = bq.reshape(1, 512)
    colsum = _attention_colsum(x, xt, Wq, Wk, bq_row)
    aux = jnp.concatenate([
        jnp.pad(Wc.T, ((0, 0), (0, 1))).reshape(480),
        bc, jnp.full((1,), -1e30, jnp.float32)])
    out16 = _build_sc_select()(colsum, aux)
    return out16[:1]
